# baseline (speedup 1.0000x reference)
import functools

import jax
import jax.numpy as jnp
from jax import lax
from jax.experimental import pallas as pl
from jax.experimental.pallas import tpu as pltpu
from jax.experimental.pallas import tpu_sc as plsc

B = 4096
H = 50
D = 64
NC = 2
NS = 16
NW = NC * NS
BAGS_PER_W = B // NW
CHUNK_BAGS = 2
ROWS = CHUNK_BAGS * H
ROWS_PAD = 104
N_CHUNKS = BAGS_PER_W // CHUNK_BAGS
LANES = 16
DCH = D // LANES


def _bag_sums(idx3, emb):
    mesh = plsc.VectorSubcoreMesh(core_axis_name="c", subcore_axis_name="s")

    @functools.partial(
        pl.kernel,
        out_type=jax.ShapeDtypeStruct((B, D), jnp.float32),
        mesh=mesh,
        scratch_types=[
            pltpu.VMEM((N_CHUNKS, ROWS_PAD), jnp.int32),
            pltpu.VMEM((ROWS_PAD, D), jnp.float32),
            pltpu.VMEM((ROWS_PAD, D), jnp.float32),
            pltpu.VMEM((BAGS_PER_W, D), jnp.float32),
            pltpu.SemaphoreType.DMA,
            pltpu.SemaphoreType.DMA,
        ],
        compiler_params=pltpu.CompilerParams(use_tc_tiling_on_sc=False),
    )
    def k(idx_hbm, emb_hbm, out_hbm, idx_v, buf0, buf1, out_v, sem0, sem1):
        wid = lax.axis_index("s") * NC + lax.axis_index("c")
        pltpu.sync_copy(idx_hbm.at[wid], idx_v)
        bufs = (buf0, buf1)
        sems = (sem0, sem1)

        pltpu.async_copy(emb_hbm.at[idx_v.at[0]], bufs[0], sems[0])

        def chunk_pair(jj, carry):
            j = jj * 2
            for p in range(2):
                kk = j + p
                pltpu.make_async_copy(
                    emb_hbm.at[idx_v.at[kk]], bufs[p], sems[p]
                ).wait()

                @pl.when(kk + 1 < N_CHUNKS)
                def _():
                    pltpu.async_copy(
                        emb_hbm.at[idx_v.at[kk + 1]], bufs[1 - p], sems[1 - p]
                    )

                for b2 in range(CHUNK_BAGS):
                    def body(r, acc, _b2=b2, _p=p):
                        row = _b2 * H + r
                        return tuple(
                            acc[c] + bufs[_p][row, pl.ds(c * LANES, LANES)]
                            for c in range(DCH)
                        )

                    acc = lax.fori_loop(
                        0, H, body,
                        tuple(jnp.zeros((LANES,), jnp.float32) for _ in range(DCH)),
                    )
                    bag = kk * CHUNK_BAGS + b2
                    for c in range(DCH):
                        out_v[bag, pl.ds(c * LANES, LANES)] = acc[c]
            return carry

        lax.fori_loop(0, N_CHUNKS // 2, chunk_pair, 0)
        pltpu.sync_copy(out_v, out_hbm.at[pl.ds(wid * BAGS_PER_W, BAGS_PER_W)])

    return k(idx3, emb)


def _head(x, w, bias):
    alpha = 1.6732632423543772
    scale = 1.0507009873554805

    def body(x_ref, w_ref, b_ref, o_ref):
        xv = x_ref[...]
        xv = scale * jnp.where(xv > 0, xv, alpha * (jnp.exp(xv) - 1.0))
        o_ref[...] = (
            lax.dot_general(
                xv, w_ref[...], (((1,), (1,)), ((), ())),
                preferred_element_type=jnp.float32,
            )
            + b_ref[...]
        )

    return pl.pallas_call(
        body,
        out_shape=jax.ShapeDtypeStruct((B, D), jnp.float32),
    )(x, w, bias)


def kernel(input_text, emb, W, b):
    idx = input_text.astype(jnp.int32).reshape(NW, N_CHUNKS, ROWS)
    pad = jnp.zeros((NW, N_CHUNKS, ROWS_PAD - ROWS), jnp.int32)
    idx3 = jnp.concatenate([idx, pad], axis=-1)
    sums = _bag_sums(idx3, emb)
    return _head(sums, W, b.reshape(1, D))


# speedup vs baseline: 1.2508x; 1.2508x over previous
import functools

import jax
import jax.numpy as jnp
from jax import lax
from jax.experimental import pallas as pl
from jax.experimental.pallas import tpu as pltpu
from jax.experimental.pallas import tpu_sc as plsc

B = 4096
H = 50
D = 64
NC = 2
NS = 16
NW = NC * NS
BAGS_PER_W = B // NW
LANES = 16
DCH = D // LANES
RG = 10
NG = H // RG
CHUNK = 4
ROWS = CHUNK * H
NCHUNK = BAGS_PER_W // CHUNK
NBUF = 4


def _bag_sums(idx_flat, emb):
    mesh = plsc.VectorSubcoreMesh(core_axis_name="c", subcore_axis_name="s")

    @functools.partial(
        pl.kernel,
        out_type=jax.ShapeDtypeStruct((B * D,), jnp.float32),
        mesh=mesh,
        scratch_types=[
            pltpu.VMEM((BAGS_PER_W * H,), jnp.int32),
            pltpu.VMEM((ROWS, D), jnp.float32),
            pltpu.VMEM((ROWS, D), jnp.float32),
            pltpu.VMEM((ROWS, D), jnp.float32),
            pltpu.VMEM((ROWS, D), jnp.float32),
            pltpu.VMEM((BAGS_PER_W * D,), jnp.float32),
            pltpu.SemaphoreType.DMA,
            pltpu.SemaphoreType.DMA,
            pltpu.SemaphoreType.DMA,
            pltpu.SemaphoreType.DMA,
        ],
        compiler_params=pltpu.CompilerParams(use_tc_tiling_on_sc=False),
    )
    def k(idx_hbm, emb_hbm, out_hbm, idx_v, b0, b1, b2, b3, out_v,
          s0, s1, s2, s3):
        wid = lax.axis_index("s") * NC + lax.axis_index("c")
        nidx = BAGS_PER_W * H
        pltpu.sync_copy(idx_hbm.at[pl.ds(wid * nidx, nidx)], idx_v)
        bufs = (b0, b1, b2, b3)
        sems = (s0, s1, s2, s3)

        def off(c):
            return idx_v.at[pl.ds(c * ROWS, ROWS)]

        for c in range(NBUF - 1):
            pltpu.async_copy(emb_hbm.at[off(c)], bufs[c], sems[c])

        def group(g, carry):
            for bslot in range(NBUF):
                c = g * NBUF + bslot
                pltpu.make_async_copy(
                    emb_hbm.at[off(c)], bufs[bslot], sems[bslot]
                ).wait()

                @pl.when(c + NBUF - 1 < NCHUNK)
                def _():
                    nxt = (bslot + NBUF - 1) % NBUF
                    pltpu.async_copy(
                        emb_hbm.at[off(c + NBUF - 1)], bufs[nxt], sems[nxt]
                    )

                for p in range(CHUNK):
                    def body(r, acc, _b=bslot, _p=p):
                        base = _p * H + r * RG
                        for rr in range(RG):
                            acc = tuple(
                                acc[ch]
                                + bufs[_b][base + rr, pl.ds(ch * LANES, LANES)]
                                for ch in range(DCH)
                            )
                        return acc

                    acc = lax.fori_loop(
                        0, NG, body,
                        tuple(
                            jnp.zeros((LANES,), jnp.float32)
                            for _ in range(DCH)
                        ),
                    )
                    bag = c * CHUNK + p
                    for ch in range(DCH):
                        out_v[pl.ds(bag * D + ch * LANES, LANES)] = acc[ch]
            return carry

        lax.fori_loop(0, NCHUNK // NBUF, group, 0)
        pltpu.sync_copy(
            out_v, out_hbm.at[pl.ds(wid * BAGS_PER_W * D, BAGS_PER_W * D)]
        )

    return k(idx_flat, emb)


def _head(x, w, bias):
    alpha = 1.6732632423543772
    scale = 1.0507009873554805

    def body(x_ref, w_ref, b_ref, o_ref):
        xv = x_ref[...]
        xv = scale * jnp.where(xv > 0, xv, alpha * (jnp.exp(xv) - 1.0))
        o_ref[...] = (
            lax.dot_general(
                xv, w_ref[...], (((1,), (1,)), ((), ())),
                preferred_element_type=jnp.float32,
            )
            + b_ref[...]
        )

    return pl.pallas_call(
        body,
        out_shape=jax.ShapeDtypeStruct((B, D), jnp.float32),
    )(x, w, bias)


def kernel(input_text, emb, W, b):
    idx_flat = input_text.astype(jnp.int32).reshape(-1)
    sums = _bag_sums(idx_flat, emb).reshape(B, D)
    return _head(sums, W, b.reshape(1, D))
